# baseline (speedup 1.0000x reference)
import functools

import jax
import jax.numpy as jnp
from jax.experimental import pallas as pl
from jax.experimental.pallas import tpu as pltpu

_LANE = 128
_MIB = 1 << 20


def _round_up(v, m):
    return ((v + m - 1) // m) * m


def _fused_gcn_body(x_ref, w_ref, b_ref, *rest):
    adj_refs = rest[:-2]
    out_ref, h_ref = rest[-2:]
    kc = h_ref.shape[0] // len(adj_refs)

    @pl.when(pl.program_id(1) == 0)
    def _():
        xv = x_ref[...].astype(jnp.bfloat16)
        wv = w_ref[...].astype(jnp.bfloat16)
        h = jax.lax.dot_general(
            xv, wv, (((1,), (1,)), ((), ())),
            preferred_element_type=jnp.float32)
        h_ref[...] = (h + b_ref[...]).astype(h_ref.dtype)

    acc = jnp.dot(adj_refs[0][...], h_ref[pl.ds(0, kc), :],
                  preferred_element_type=jnp.float32)
    for s in range(1, len(adj_refs)):
        acc += jnp.dot(adj_refs[s][...], h_ref[pl.ds(s * kc, kc), :],
                       preferred_element_type=jnp.float32)
    out_ref[...] = jnp.maximum(acc, 0.0).astype(out_ref.dtype)


@functools.partial(jax.jit, static_argnames=("block_m", "n_streams"))
def _gcn_forward(x, adj, weight, bias, block_m=1024, n_streams=2):
    n, f_in = x.shape
    f_out = weight.shape[0]
    n_p = _round_up(n, _LANE)
    f_in_p = _round_up(f_in, _LANE)
    f_out_p = _round_up(f_out, _LANE)

    if adj.shape[0] != n_p:
        adj_w = jnp.zeros((n_p, n_p), adj.dtype).at[:n, :n].set(adj)
    else:
        adj_w = adj
    if x.shape != (n_p, f_in_p):
        x_w = jnp.zeros((n_p, f_in_p), x.dtype).at[:n, :f_in].set(x)
    else:
        x_w = x
    if weight.shape != (f_out_p, f_in_p):
        w_w = jnp.zeros((f_out_p, f_in_p), weight.dtype)
        w_w = w_w.at[:f_out, :f_in].set(weight)
    else:
        w_w = weight
    b_p = jnp.zeros((1, f_out_p), jnp.float32)
    if bias is not None:
        b_p = b_p.at[0, :f_out].set(bias.astype(jnp.float32))

    bm = block_m
    while n_p % (2 * bm) and bm > _LANE:
        bm //= 2
    nj = n_p // (2 * bm)
    ns = n_streams
    while n_p % (ns * _LANE) and ns > 1:
        ns //= 2
    kc = n_p // ns

    def _adj_spec(s):
        return pl.BlockSpec((bm, kc), lambda c, j, s=s: (c * nj + j, s))

    out_p = pl.pallas_call(
        _fused_gcn_body,
        out_shape=jax.ShapeDtypeStruct((n_p, f_out_p), x.dtype),
        grid=(2, nj),
        in_specs=[
            pl.BlockSpec((n_p, f_in_p), lambda c, j: (0, 0)),
            pl.BlockSpec((f_out_p, f_in_p), lambda c, j: (0, 0)),
            pl.BlockSpec((1, f_out_p), lambda c, j: (0, 0)),
        ] + [_adj_spec(s) for s in range(ns)],
        out_specs=pl.BlockSpec((bm, f_out_p), lambda c, j: (c * nj + j, 0)),
        scratch_shapes=[pltpu.VMEM((n_p, f_out_p), jnp.bfloat16)],
        compiler_params=pltpu.CompilerParams(
            dimension_semantics=("parallel", "arbitrary"),
            vmem_limit_bytes=60 * _MIB),
    )(x_w, w_w, b_p, *([adj_w] * ns))

    if (n_p, f_out_p) != (n, f_out):
        return out_p[:n, :f_out]
    return out_p


def kernel(x, adj, weight, bias):
    out = _gcn_forward(x, adj, weight, bias)
    return out, adj


# speedup vs baseline: 1.0202x; 1.0202x over previous
import functools

import jax
import jax.numpy as jnp
from jax.experimental import pallas as pl
from jax.experimental.pallas import tpu as pltpu

_LANE = 128
_MIB = 1 << 20


def _round_up(v, m):
    return ((v + m - 1) // m) * m


def _fused_gcn_body(x_ref, w_ref, b_ref, adj_ref, out_ref, h_ref):
    @pl.when(pl.program_id(1) == 0)
    def _():
        xv = x_ref[...].astype(jnp.bfloat16)
        wv = w_ref[...].astype(jnp.bfloat16)
        h = jax.lax.dot_general(
            xv, wv, (((1,), (1,)), ((), ())),
            preferred_element_type=jnp.float32)
        h_ref[...] = (h + b_ref[...]).astype(h_ref.dtype)

    acc = jnp.dot(adj_ref[...], h_ref[...], preferred_element_type=jnp.float32)
    out_ref[...] = jnp.maximum(acc, 0.0).astype(out_ref.dtype)


@functools.partial(jax.jit, static_argnames=("block_m",))
def _gcn_forward(x, adj, weight, bias, block_m=1024):
    n, f_in = x.shape
    f_out = weight.shape[0]
    n_p = _round_up(n, _LANE)
    f_in_p = _round_up(f_in, _LANE)
    f_out_p = _round_up(f_out, _LANE)

    if adj.shape[0] != n_p:
        adj_w = jnp.zeros((n_p, n_p), adj.dtype).at[:n, :n].set(adj)
    else:
        adj_w = adj
    if x.shape != (n_p, f_in_p):
        x_w = jnp.zeros((n_p, f_in_p), x.dtype).at[:n, :f_in].set(x)
    else:
        x_w = x
    if weight.shape != (f_out_p, f_in_p):
        w_w = jnp.zeros((f_out_p, f_in_p), weight.dtype)
        w_w = w_w.at[:f_out, :f_in].set(weight)
    else:
        w_w = weight
    b_p = jnp.zeros((1, f_out_p), jnp.float32)
    if bias is not None:
        b_p = b_p.at[0, :f_out].set(bias.astype(jnp.float32))

    bm = block_m
    while n_p % (2 * bm) and bm > _LANE:
        bm //= 2
    nj = n_p // (2 * bm)

    out_p = pl.pallas_call(
        _fused_gcn_body,
        out_shape=jax.ShapeDtypeStruct((n_p, f_out_p), x.dtype),
        grid=(2, nj),
        in_specs=[
            pl.BlockSpec((n_p, f_in_p), lambda c, j: (0, 0)),
            pl.BlockSpec((f_out_p, f_in_p), lambda c, j: (0, 0)),
            pl.BlockSpec((1, f_out_p), lambda c, j: (0, 0)),
            pl.BlockSpec((bm, n_p), lambda c, j: (c * nj + j, 0)),
        ],
        out_specs=pl.BlockSpec((bm, f_out_p), lambda c, j: (c * nj + j, 0)),
        scratch_shapes=[pltpu.VMEM((n_p, f_out_p), jnp.bfloat16)],
        compiler_params=pltpu.CompilerParams(
            dimension_semantics=("parallel", "arbitrary"),
            vmem_limit_bytes=60 * _MIB),
    )(x_w, w_w, b_p, adj_w)

    if (n_p, f_out_p) != (n, f_out):
        return out_p[:n, :f_out]
    return out_p


def kernel(x, adj, weight, bias):
    out = _gcn_forward(x, adj, weight, bias)
    return out, adj
